# baseline (speedup 1.0000x reference)
import functools
import jax
import jax.numpy as jnp
from jax import lax
from jax.experimental import pallas as pl
from jax.experimental.pallas import tpu as pltpu
from jax.experimental.pallas import tpu_sc as plsc

_R = 4
_N = 4096
_F = 64
_H = 64
_O = 32
_BJ = 512
_NJ = _N // _BJ


def _sc_gather(table, idx):
    info = plsc.get_sparse_core_info()
    nw = info.num_cores * info.num_subcores
    bpw = idx.shape[0] // nw
    mesh = plsc.VectorSubcoreMesh(core_axis_name="c", subcore_axis_name="s")

    @functools.partial(
        pl.kernel,
        mesh=mesh,
        out_type=jax.ShapeDtypeStruct((idx.shape[0], table.shape[1]), table.dtype),
        scratch_types=[
            pltpu.VMEM((bpw,), jnp.int32),
            pltpu.VMEM((bpw, table.shape[1]), table.dtype),
            pltpu.SemaphoreType.DMA,
        ],
    )
    def k(table_hbm, idx_hbm, out_hbm, idx_v, rows_v, sem):
        wid = lax.axis_index("s") * info.num_cores + lax.axis_index("c")
        base = wid * bpw
        pltpu.sync_copy(idx_hbm.at[pl.ds(base, bpw)], idx_v)
        pltpu.async_copy(table_hbm.at[idx_v], rows_v, sem).wait()
        pltpu.sync_copy(rows_v, out_hbm.at[pl.ds(base, bpw)])

    return k(table, idx)


def _pass1_body(adj_ref, feat2_ref, par_ref, w1_ref, u1_ref, dout_ref, din_ref):
    jb = pl.program_id(1)
    a = adj_ref[0]
    m = (a != 0.0).astype(jnp.float32)
    ones_n = jnp.ones((_N, 1), jnp.float32)
    ones_bj = jnp.ones((_BJ, 1), jnp.float32)
    rowsum = lax.dot_general(
        m, ones_n, (((1,), (0,)), ((), ())),
        preferred_element_type=jnp.float32)
    dout_ref[0, 0, :] = rowsum[:, 0]

    @pl.when(jb == 0)
    def _():
        din_ref[0, 0, :] = jnp.zeros((_N,), jnp.float32)
        u1_ref[0] = jnp.zeros((_N, _H), jnp.float32)

    colsum = lax.dot_general(
        m, ones_bj, (((0,), (0,)), ((), ())),
        preferred_element_type=jnp.float32)
    din_ref[0, 0, :] += colsum[:, 0]
    s = lax.rsqrt(jnp.maximum(rowsum, 1.0))
    f2 = feat2_ref[...]
    feat = jnp.where(par_ref[...] != 0, f2[:, _F:2 * _F], f2[:, 0:_F])
    fw = jnp.dot(feat, w1_ref[0], preferred_element_type=jnp.float32)
    fw = fw * s
    u1_ref[0] += lax.dot_general(
        m, fw, (((0,), (0,)), ((), ())), preferred_element_type=jnp.float32)


def _pass1(adj, feat2, parity, w1):
    return pl.pallas_call(
        _pass1_body,
        grid=(_R, _NJ),
        in_specs=[
            pl.BlockSpec((1, _BJ, _N), lambda r, j: (r, j, 0)),
            pl.BlockSpec((_BJ, 2 * _F), lambda r, j: (j, 0)),
            pl.BlockSpec((_BJ, 1), lambda r, j: (j, 0)),
            pl.BlockSpec((1, _F, _H), lambda r, j: (r, 0, 0)),
        ],
        out_specs=[
            pl.BlockSpec((1, _N, _H), lambda r, j: (r, 0, 0)),
            pl.BlockSpec((1, 1, _BJ), lambda r, j: (r, 0, j)),
            pl.BlockSpec((1, 1, _N), lambda r, j: (r, 0, 0)),
        ],
        out_shape=[
            jax.ShapeDtypeStruct((_R, _N, _H), jnp.float32),
            jax.ShapeDtypeStruct((_R, 1, _N), jnp.float32),
            jax.ShapeDtypeStruct((_R, 1, _N), jnp.float32),
        ],
        compiler_params=pltpu.CompilerParams(
            dimension_semantics=("arbitrary", "arbitrary")),
    )(adj, feat2, parity, w1)


def _pass2_body(adj_ref, u1_ref, din_ref, dout_ref, b1_ref, w2_ref, b2_ref,
                wf_ref, bf_ref, wb_ref, bb_ref, out_ref, h1_s, x2_s, acc_s):
    r = pl.program_id(0)
    jb = pl.program_id(1)

    @pl.when((r == 0) & (jb == 0))
    def _():
        acc = jnp.zeros((_N, _H), jnp.float32)
        for rr in range(_R):
            s_in = lax.rsqrt(jnp.maximum(din_ref[rr, 0, :], 1.0))
            acc = acc + u1_ref[rr] * s_in[:, None] + b1_ref[rr][None, :]
        h1_s[...] = jnp.maximum(acc * (1.0 / _R), 0.0)
        acc_s[...] = jnp.zeros((_N, _O), jnp.float32)

    @pl.when(jb == 0)
    def _():
        x2_s[...] = jnp.dot(h1_s[...], w2_ref[0],
                            preferred_element_type=jnp.float32)

    a = adj_ref[0]
    m = (a != 0.0).astype(jnp.float32)
    s_out = lax.rsqrt(jnp.maximum(dout_ref[0, 0, :], 1.0))
    xb = x2_s[pl.ds(jb * _BJ, _BJ), :] * s_out[:, None]
    contrib = lax.dot_general(
        m, xb, (((0,), (0,)), ((), ())), preferred_element_type=jnp.float32)
    s_in = lax.rsqrt(jnp.maximum(din_ref[r, 0, :], 1.0))
    acc_s[...] += s_in[:, None] * contrib

    @pl.when((r == _R - 1) & (jb == _NJ - 1))
    def _():
        h2 = acc_s[...] * (1.0 / _R) + jnp.mean(b2_ref[...], axis=0)[None, :]

        def lstm(w, b):
            gates = lax.dot_general(
                h2, w, (((1,), (1,)), ((), ())),
                preferred_element_type=jnp.float32) + b
            sg = jax.nn.sigmoid(gates)
            i_s = sg[:, 0:16]
            g_t = jnp.tanh(gates[:, 32:48])
            o_s = sg[:, 48:64]
            return o_s * jnp.tanh(i_s * g_t)

        out_ref[:, 0:16] = lstm(wf_ref[...], bf_ref[...])
        out_ref[:, 16:32] = lstm(wb_ref[...], bb_ref[...])


def _pass2(adj, u1, din, dout, b1, w2, b2, wf, bf, wb, bb):
    return pl.pallas_call(
        _pass2_body,
        grid=(_R, _NJ),
        in_specs=[
            pl.BlockSpec((1, _BJ, _N), lambda r, j: (r, j, 0)),
            pl.BlockSpec((_R, _N, _H), lambda r, j: (0, 0, 0)),
            pl.BlockSpec((_R, 1, _N), lambda r, j: (0, 0, 0)),
            pl.BlockSpec((1, 1, _BJ), lambda r, j: (r, 0, j)),
            pl.BlockSpec((_R, _H), lambda r, j: (0, 0)),
            pl.BlockSpec((1, _H, _O), lambda r, j: (r, 0, 0)),
            pl.BlockSpec((_R, _O), lambda r, j: (0, 0)),
            pl.BlockSpec((4 * _O // 2, _O), lambda r, j: (0, 0)),
            pl.BlockSpec((1, 4 * _O // 2), lambda r, j: (0, 0)),
            pl.BlockSpec((4 * _O // 2, _O), lambda r, j: (0, 0)),
            pl.BlockSpec((1, 4 * _O // 2), lambda r, j: (0, 0)),
        ],
        out_specs=pl.BlockSpec((_N, _O), lambda r, j: (0, 0)),
        out_shape=jax.ShapeDtypeStruct((_N, _O), jnp.float32),
        scratch_shapes=[
            pltpu.VMEM((_N, _H), jnp.float32),
            pltpu.VMEM((_N, _O), jnp.float32),
            pltpu.VMEM((_N, _O), jnp.float32),
        ],
        compiler_params=pltpu.CompilerParams(
            dimension_semantics=("arbitrary", "arbitrary")),
    )(adj, u1, din, dout, b1, w2, b2, wf, bf, wb, bb)


def kernel(entity_ids, rel_adj_matrices, entity_emb, conv1_W, conv1_b,
           conv2_W, conv2_b, W_ih_f, W_hh_f, b_ih_f, b_hh_f,
           W_ih_b, W_hh_b, b_ih_b, b_hh_b):
    emb2 = entity_emb.reshape(entity_emb.shape[0] // 2, 2 * _F)
    feat2 = _sc_gather(emb2, entity_ids >> 1)
    parity = (entity_ids & 1).reshape(_N, 1)
    u1, dout, din = _pass1(rel_adj_matrices, feat2, parity, conv1_W)
    bf = (b_ih_f + b_hh_f).reshape(1, -1)
    bb = (b_ih_b + b_hh_b).reshape(1, -1)
    return _pass2(rel_adj_matrices, u1, din, dout, conv1_b, conv2_W, conv2_b,
                  W_ih_f, bf, W_ih_b, bb)


# speedup vs baseline: 1.4315x; 1.4315x over previous
import functools
import jax
import jax.numpy as jnp
from jax import lax
from jax.experimental import pallas as pl
from jax.experimental.pallas import tpu as pltpu
from jax.experimental.pallas import tpu_sc as plsc

_R = 4
_N = 4096
_F = 64
_H = 64
_O = 32
_BJ = 512
_NJ = _N // _BJ
_HA = _H + 1


def _sc_gather(table, idx):
    info = plsc.get_sparse_core_info()
    nw = info.num_cores * info.num_subcores
    bpw = idx.shape[0] // nw
    mesh = plsc.VectorSubcoreMesh(core_axis_name="c", subcore_axis_name="s")

    @functools.partial(
        pl.kernel,
        mesh=mesh,
        out_type=jax.ShapeDtypeStruct((idx.shape[0], table.shape[1]), table.dtype),
        scratch_types=[
            pltpu.VMEM((bpw,), jnp.int32),
            pltpu.VMEM((bpw, table.shape[1]), table.dtype),
            pltpu.SemaphoreType.DMA,
        ],
    )
    def k(table_hbm, idx_hbm, out_hbm, idx_v, rows_v, sem):
        wid = lax.axis_index("s") * info.num_cores + lax.axis_index("c")
        base = wid * bpw
        pltpu.sync_copy(idx_hbm.at[pl.ds(base, bpw)], idx_v)
        pltpu.async_copy(table_hbm.at[idx_v], rows_v, sem).wait()
        pltpu.sync_copy(rows_v, out_hbm.at[pl.ds(base, bpw)])

    return k(table, idx)


def _pass1_body(adj_ref, feat2_ref, par_ref, w1_ref, u1_ref, dout_ref):
    jb = pl.program_id(1)
    a = adj_ref[0]
    m = (a != 0.0).astype(jnp.float32)
    rowsum = lax.dot_general(
        m, jnp.ones((_N, 1), jnp.float32), (((1,), (0,)), ((), ())),
        preferred_element_type=jnp.float32)
    dout_ref[0, 0, :] = lax.transpose(rowsum, (1, 0))[0]
    s = lax.rsqrt(jnp.maximum(rowsum, 1.0))

    @pl.when(jb == 0)
    def _():
        u1_ref[0] = jnp.zeros((_HA, _N), jnp.float32)

    f2 = feat2_ref[...]
    feat = jnp.where(par_ref[...] != 0, f2[:, _F:2 * _F], f2[:, 0:_F])
    fwT = lax.dot_general(
        w1_ref[0], feat * s, (((0,), (1,)), ((), ())),
        preferred_element_type=jnp.float32)
    fwT_aug = jnp.concatenate(
        [fwT, jnp.ones((1, _BJ), jnp.float32)], axis=0)
    u1_ref[0] += lax.dot_general(
        fwT_aug, m, (((1,), (0,)), ((), ())),
        preferred_element_type=jnp.float32)


def _pass1(adj, feat2, parity, w1):
    return pl.pallas_call(
        _pass1_body,
        grid=(_R, _NJ),
        in_specs=[
            pl.BlockSpec((1, _BJ, _N), lambda r, j: (r, j, 0)),
            pl.BlockSpec((_BJ, 2 * _F), lambda r, j: (j, 0)),
            pl.BlockSpec((_BJ, 1), lambda r, j: (j, 0)),
            pl.BlockSpec((1, _F, _H), lambda r, j: (r, 0, 0)),
        ],
        out_specs=[
            pl.BlockSpec((1, _HA, _N), lambda r, j: (r, 0, 0)),
            pl.BlockSpec((1, 1, _BJ), lambda r, j: (r, 0, j)),
        ],
        out_shape=[
            jax.ShapeDtypeStruct((_R, _HA, _N), jnp.float32),
            jax.ShapeDtypeStruct((_R, 1, _N), jnp.float32),
        ],
        compiler_params=pltpu.CompilerParams(
            dimension_semantics=("arbitrary", "arbitrary")),
    )(adj, feat2, parity, w1)


def _pass2_body(adj_ref, u1_ref, dout_ref, b1_ref, w2_ref, b2_ref,
                wf_ref, bf_ref, wb_ref, bb_ref, out_ref,
                h1_s, x2_s, acc_s, sin_s):
    r = pl.program_id(0)
    jb = pl.program_id(1)

    @pl.when((r == 0) & (jb == 0))
    def _():
        acc = jnp.zeros((_H, _N), jnp.float32)
        for rr in range(_R):
            s_in = lax.rsqrt(jnp.maximum(u1_ref[rr, _H, :], 1.0))
            sin_s[rr, :] = s_in
            acc = (acc + u1_ref[rr, 0:_H, :] * s_in[None, :]
                   + b1_ref[rr][:, None])
        h1_s[...] = jnp.maximum(acc * (1.0 / _R), 0.0)
        acc_s[...] = jnp.zeros((_O, _N), jnp.float32)

    @pl.when(jb == 0)
    def _():
        x2_s[...] = lax.dot_general(
            w2_ref[0], h1_s[...], (((0,), (0,)), ((), ())),
            preferred_element_type=jnp.float32)

    a = adj_ref[0]
    m = (a != 0.0).astype(jnp.float32)
    s_out = lax.rsqrt(jnp.maximum(dout_ref[0, 0, :], 1.0))
    xbT = x2_s[:, pl.ds(jb * _BJ, _BJ)] * s_out[None, :]
    contrib = lax.dot_general(
        xbT, m, (((1,), (0,)), ((), ())),
        preferred_element_type=jnp.float32)
    acc_s[...] += contrib * sin_s[r, :][None, :]

    @pl.when((r == _R - 1) & (jb == _NJ - 1))
    def _():
        h2 = acc_s[...] * (1.0 / _R) + jnp.mean(b2_ref[...], axis=0)[:, None]

        def lstm(w, b):
            gatesT = lax.dot_general(
                w, h2, (((1,), (0,)), ((), ())),
                preferred_element_type=jnp.float32) + b[0][:, None]
            i_s = jax.nn.sigmoid(gatesT[0:16, :])
            g_t = jnp.tanh(gatesT[32:48, :])
            o_s = jax.nn.sigmoid(gatesT[48:64, :])
            return o_s * jnp.tanh(i_s * g_t)

        outT = jnp.concatenate(
            [lstm(wf_ref[...], bf_ref[...]), lstm(wb_ref[...], bb_ref[...])],
            axis=0)
        out_ref[...] = lax.transpose(outT, (1, 0))


def _pass2(adj, u1, dout, b1, w2, b2, wf, bf, wb, bb):
    return pl.pallas_call(
        _pass2_body,
        grid=(_R, _NJ),
        in_specs=[
            pl.BlockSpec((1, _BJ, _N), lambda r, j: (r, j, 0)),
            pl.BlockSpec((_R, _HA, _N), lambda r, j: (0, 0, 0)),
            pl.BlockSpec((1, 1, _BJ), lambda r, j: (r, 0, j)),
            pl.BlockSpec((_R, _H), lambda r, j: (0, 0)),
            pl.BlockSpec((1, _H, _O), lambda r, j: (r, 0, 0)),
            pl.BlockSpec((_R, _O), lambda r, j: (0, 0)),
            pl.BlockSpec((4 * _O // 2, _O), lambda r, j: (0, 0)),
            pl.BlockSpec((1, 4 * _O // 2), lambda r, j: (0, 0)),
            pl.BlockSpec((4 * _O // 2, _O), lambda r, j: (0, 0)),
            pl.BlockSpec((1, 4 * _O // 2), lambda r, j: (0, 0)),
        ],
        out_specs=pl.BlockSpec((_N, _O), lambda r, j: (0, 0)),
        out_shape=jax.ShapeDtypeStruct((_N, _O), jnp.float32),
        scratch_shapes=[
            pltpu.VMEM((_H, _N), jnp.float32),
            pltpu.VMEM((_O, _N), jnp.float32),
            pltpu.VMEM((_O, _N), jnp.float32),
            pltpu.VMEM((_R, _N), jnp.float32),
        ],
        compiler_params=pltpu.CompilerParams(
            dimension_semantics=("arbitrary", "arbitrary")),
    )(adj, u1, dout, b1, w2, b2, wf, bf, wb, bb)


def kernel(entity_ids, rel_adj_matrices, entity_emb, conv1_W, conv1_b,
           conv2_W, conv2_b, W_ih_f, W_hh_f, b_ih_f, b_hh_f,
           W_ih_b, W_hh_b, b_ih_b, b_hh_b):
    emb2 = entity_emb.reshape(entity_emb.shape[0] // 2, 2 * _F)
    feat2 = _sc_gather(emb2, entity_ids >> 1)
    parity = (entity_ids & 1).reshape(_N, 1)
    u1, dout = _pass1(rel_adj_matrices, feat2, parity, conv1_W)
    bf = (b_ih_f + b_hh_f).reshape(1, -1)
    bb = (b_ih_b + b_hh_b).reshape(1, -1)
    return _pass2(rel_adj_matrices, u1, dout, conv1_b, conv2_W, conv2_b,
                  W_ih_f, bf, W_ih_b, bb)


# speedup vs baseline: 1.6062x; 1.1221x over previous
import functools
import jax
import jax.numpy as jnp
from jax import lax
from jax.experimental import pallas as pl
from jax.experimental.pallas import tpu as pltpu
from jax.experimental.pallas import tpu_sc as plsc

_R = 4
_N = 4096
_F = 64
_H = 64
_O = 32
_BJ = 1024
_NJ = _N // _BJ
_HA = _H + 1


def _sc_gather(table, idx):
    info = plsc.get_sparse_core_info()
    nw = info.num_cores * info.num_subcores
    bpw = idx.shape[0] // nw
    mesh = plsc.VectorSubcoreMesh(core_axis_name="c", subcore_axis_name="s")

    @functools.partial(
        pl.kernel,
        mesh=mesh,
        out_type=jax.ShapeDtypeStruct((idx.shape[0], table.shape[1]), table.dtype),
        scratch_types=[
            pltpu.VMEM((bpw,), jnp.int32),
            pltpu.VMEM((bpw, table.shape[1]), table.dtype),
            pltpu.SemaphoreType.DMA,
        ],
    )
    def k(table_hbm, idx_hbm, out_hbm, idx_v, rows_v, sem):
        wid = lax.axis_index("s") * info.num_cores + lax.axis_index("c")
        base = wid * bpw
        pltpu.sync_copy(idx_hbm.at[pl.ds(base, bpw)], idx_v)
        pltpu.async_copy(table_hbm.at[idx_v], rows_v, sem).wait()
        pltpu.sync_copy(rows_v, out_hbm.at[pl.ds(base, bpw)])

    return k(table, idx)


def _pass1_body(adj_ref, feat2_ref, par_ref, w1_ref, u1_ref, dout_ref):
    jb = pl.program_id(1)
    a = adj_ref[0]
    m = (a != 0.0).astype(jnp.float32)
    rowsum = jnp.sum(m, axis=1, keepdims=True)
    dout_ref[0, 0, :] = lax.transpose(rowsum, (1, 0))[0]
    s = lax.rsqrt(jnp.maximum(rowsum, 1.0))

    @pl.when(jb == 0)
    def _():
        u1_ref[0] = jnp.zeros((_HA, _N), jnp.float32)

    f2 = feat2_ref[...]
    feat = jnp.where(par_ref[...] != 0, f2[:, _F:2 * _F], f2[:, 0:_F])
    fwT = lax.dot_general(
        w1_ref[0], feat * s, (((0,), (1,)), ((), ())),
        preferred_element_type=jnp.float32)
    fwT_aug = jnp.concatenate(
        [fwT, jnp.ones((1, _BJ), jnp.float32)], axis=0)
    u1_ref[0] += lax.dot_general(
        fwT_aug, m, (((1,), (0,)), ((), ())),
        preferred_element_type=jnp.float32)


def _pass1(adj, feat2, parity, w1):
    return pl.pallas_call(
        _pass1_body,
        grid=(_R, _NJ),
        in_specs=[
            pl.BlockSpec((1, _BJ, _N), lambda r, j: (r, j, 0)),
            pl.BlockSpec((_BJ, 2 * _F), lambda r, j: (j, 0)),
            pl.BlockSpec((_BJ, 1), lambda r, j: (j, 0)),
            pl.BlockSpec((1, _F, _H), lambda r, j: (r, 0, 0)),
        ],
        out_specs=[
            pl.BlockSpec((1, _HA, _N), lambda r, j: (r, 0, 0)),
            pl.BlockSpec((1, 1, _BJ), lambda r, j: (r, 0, j)),
        ],
        out_shape=[
            jax.ShapeDtypeStruct((_R, _HA, _N), jnp.float32),
            jax.ShapeDtypeStruct((_R, 1, _N), jnp.float32),
        ],
        compiler_params=pltpu.CompilerParams(
            dimension_semantics=("arbitrary", "arbitrary")),
    )(adj, feat2, parity, w1)


def _pass2_body(adj_ref, u1_ref, dout_ref, b1_ref, w2_ref, b2_ref,
                wf_ref, bf_ref, wb_ref, bb_ref, out_ref,
                h1_s, x2_s, acc_s, sin_s):
    r = pl.program_id(0)
    jb = pl.program_id(1)

    @pl.when((r == 0) & (jb == 0))
    def _():
        acc = jnp.zeros((_H, _N), jnp.float32)
        for rr in range(_R):
            s_in = lax.rsqrt(jnp.maximum(u1_ref[rr, _H, :], 1.0))
            sin_s[rr, :] = s_in
            acc = (acc + u1_ref[rr, 0:_H, :] * s_in[None, :]
                   + b1_ref[rr][:, None])
        h1_s[...] = jnp.maximum(acc * (1.0 / _R), 0.0)
        acc_s[...] = jnp.zeros((_O, _N), jnp.float32)

    @pl.when(jb == 0)
    def _():
        x2_s[...] = lax.dot_general(
            w2_ref[0], h1_s[...], (((0,), (0,)), ((), ())),
            preferred_element_type=jnp.float32)

    a = adj_ref[0]
    m = (a != 0.0).astype(jnp.float32)
    s_out = lax.rsqrt(jnp.maximum(dout_ref[0, 0, :], 1.0))
    xbT = x2_s[:, pl.ds(jb * _BJ, _BJ)] * s_out[None, :]
    contrib = lax.dot_general(
        xbT, m, (((1,), (0,)), ((), ())),
        preferred_element_type=jnp.float32)
    acc_s[...] += contrib * sin_s[r, :][None, :]

    @pl.when((r == _R - 1) & (jb == _NJ - 1))
    def _():
        h2 = acc_s[...] * (1.0 / _R) + jnp.mean(b2_ref[...], axis=0)[:, None]

        def lstm(w, b):
            gatesT = lax.dot_general(
                w, h2, (((1,), (0,)), ((), ())),
                preferred_element_type=jnp.float32) + b[0][:, None]
            i_s = jax.nn.sigmoid(gatesT[0:16, :])
            g_t = jnp.tanh(gatesT[32:48, :])
            o_s = jax.nn.sigmoid(gatesT[48:64, :])
            return o_s * jnp.tanh(i_s * g_t)

        outT = jnp.concatenate(
            [lstm(wf_ref[...], bf_ref[...]), lstm(wb_ref[...], bb_ref[...])],
            axis=0)
        out_ref[...] = lax.transpose(outT, (1, 0))


def _pass2(adj, u1, dout, b1, w2, b2, wf, bf, wb, bb):
    return pl.pallas_call(
        _pass2_body,
        grid=(_R, _NJ),
        in_specs=[
            pl.BlockSpec((1, _BJ, _N), lambda r, j: (r, j, 0)),
            pl.BlockSpec((_R, _HA, _N), lambda r, j: (0, 0, 0)),
            pl.BlockSpec((1, 1, _BJ), lambda r, j: (r, 0, j)),
            pl.BlockSpec((_R, _H), lambda r, j: (0, 0)),
            pl.BlockSpec((1, _H, _O), lambda r, j: (r, 0, 0)),
            pl.BlockSpec((_R, _O), lambda r, j: (0, 0)),
            pl.BlockSpec((4 * _O // 2, _O), lambda r, j: (0, 0)),
            pl.BlockSpec((1, 4 * _O // 2), lambda r, j: (0, 0)),
            pl.BlockSpec((4 * _O // 2, _O), lambda r, j: (0, 0)),
            pl.BlockSpec((1, 4 * _O // 2), lambda r, j: (0, 0)),
        ],
        out_specs=pl.BlockSpec((_N, _O), lambda r, j: (0, 0)),
        out_shape=jax.ShapeDtypeStruct((_N, _O), jnp.float32),
        scratch_shapes=[
            pltpu.VMEM((_H, _N), jnp.float32),
            pltpu.VMEM((_O, _N), jnp.float32),
            pltpu.VMEM((_O, _N), jnp.float32),
            pltpu.VMEM((_R, _N), jnp.float32),
        ],
        compiler_params=pltpu.CompilerParams(
            dimension_semantics=("arbitrary", "arbitrary")),
    )(adj, u1, dout, b1, w2, b2, wf, bf, wb, bb)


def kernel(entity_ids, rel_adj_matrices, entity_emb, conv1_W, conv1_b,
           conv2_W, conv2_b, W_ih_f, W_hh_f, b_ih_f, b_hh_f,
           W_ih_b, W_hh_b, b_ih_b, b_hh_b):
    emb2 = entity_emb.reshape(entity_emb.shape[0] // 2, 2 * _F)
    feat2 = _sc_gather(emb2, entity_ids >> 1)
    parity = (entity_ids & 1).reshape(_N, 1)
    u1, dout = _pass1(rel_adj_matrices, feat2, parity, conv1_W)
    bf = (b_ih_f + b_hh_f).reshape(1, -1)
    bb = (b_ih_b + b_hh_b).reshape(1, -1)
    return _pass2(rel_adj_matrices, u1, dout, conv1_b, conv2_W, conv2_b,
                  W_ih_f, bf, W_ih_b, bb)
